# baseline (speedup 1.0000x reference)
import functools

import jax
import jax.numpy as jnp
from jax import lax
from jax.experimental import pallas as pl
from jax.experimental.pallas import tpu as pltpu
from jax.experimental.pallas import tpu_sc as plsc

B = 16384
D = 128
R = 8192
NB = B // R

NC, NS = 2, 16
NW = NC * NS
RPW = B // NW


def _tc_body(x_ref, xui_ref):
    xui_ref[...] = jnp.sum(x_ref[0] * x_ref[1], axis=1).reshape(R // 128, 128)


def _tc_xui(inputs):
    xui2d = pl.pallas_call(
        _tc_body,
        grid=(NB,),
        in_specs=[pl.BlockSpec((2, R, D), lambda i: (0, i, 0))],
        out_specs=[pl.BlockSpec((R // 128, 128), lambda i: (i, 0))],
        out_shape=[jax.ShapeDtypeStruct((B // 128, 128), jnp.float32)],
    )(inputs)[0]
    return xui2d.reshape(B)


_sc_mesh = plsc.VectorSubcoreMesh(core_axis_name="c", subcore_axis_name="s")


@functools.partial(
    pl.kernel,
    out_type=[
        jax.ShapeDtypeStruct((B, D), jnp.float32),
        jax.ShapeDtypeStruct((B, D), jnp.float32),
    ],
    mesh=_sc_mesh,
    scratch_types=[pltpu.SemaphoreType.DMA, pltpu.SemaphoreType.DMA],
)
def _sc_copy(x_hbm, gu_hbm, gi_hbm, sem_u, sem_i):
    wid = lax.axis_index("s") * NC + lax.axis_index("c")
    base = wid * RPW
    cu = pltpu.make_async_copy(
        x_hbm.at[0, pl.ds(base, RPW), :], gu_hbm.at[pl.ds(base, RPW), :], sem_u)
    ci = pltpu.make_async_copy(
        x_hbm.at[1, pl.ds(base, RPW), :], gi_hbm.at[pl.ds(base, RPW), :], sem_i)
    cu.start()
    ci.start()
    cu.wait()
    ci.wait()


def kernel(inputs):
    xui = _tc_xui(inputs)
    gu_out, gi_out = _sc_copy(inputs)
    return (xui, gu_out, gi_out)


# speedup vs baseline: 15.3611x; 15.3611x over previous
import functools

import jax
import jax.numpy as jnp
from jax import lax
from jax.experimental import pallas as pl
from jax.experimental.pallas import tpu as pltpu
from jax.experimental.pallas import tpu_sc as plsc

B = 16384
D = 128
R = 8192
NB = B // R

NC, NS = 2, 16
NW = NC * NS
RPW = B // NW
CH = 128
NCH = RPW // CH


def _tc_body(x_ref, xui_ref):
    xui_ref[...] = jnp.sum(x_ref[0] * x_ref[1], axis=1).reshape(R // 128, 128)


def _tc_xui(inputs):
    xui2d = pl.pallas_call(
        _tc_body,
        grid=(NB,),
        in_specs=[pl.BlockSpec((2, R, D), lambda i: (0, i, 0))],
        out_specs=[pl.BlockSpec((R // 128, 128), lambda i: (i, 0))],
        out_shape=[jax.ShapeDtypeStruct((B // 128, 128), jnp.float32)],
    )(inputs)[0]
    return xui2d.reshape(B)


_sc_mesh = plsc.VectorSubcoreMesh(core_axis_name="c", subcore_axis_name="s")


@functools.partial(
    pl.kernel,
    out_type=[
        jax.ShapeDtypeStruct((B, D), jnp.float32),
        jax.ShapeDtypeStruct((B, D), jnp.float32),
    ],
    mesh=_sc_mesh,
    scratch_types=[
        pltpu.VMEM((2, CH, D), jnp.float32),
        pltpu.VMEM((2, CH, D), jnp.float32),
        pltpu.SemaphoreType.DMA,
        pltpu.SemaphoreType.DMA,
        pltpu.SemaphoreType.DMA,
    ],
)
def _sc_copy(x_hbm, gu_hbm, gi_hbm, bu, bi, sem_in0, sem_in1, sem_out):
    wid = lax.axis_index("s") * NC + lax.axis_index("c")
    base = wid * RPW
    in_sems = (sem_in0, sem_in1)

    def start_in(k):
        s = k % 2
        cu = pltpu.make_async_copy(
            x_hbm.at[0, pl.ds(base + k * CH, CH), :], bu.at[s], in_sems[s])
        ci = pltpu.make_async_copy(
            x_hbm.at[1, pl.ds(base + k * CH, CH), :], bi.at[s], in_sems[s])
        cu.start()
        ci.start()
        return (cu, ci)

    def start_out(k):
        s = k % 2
        cu = pltpu.make_async_copy(
            bu.at[s], gu_hbm.at[pl.ds(base + k * CH, CH), :], sem_out)
        ci = pltpu.make_async_copy(
            bi.at[s], gi_hbm.at[pl.ds(base + k * CH, CH), :], sem_out)
        cu.start()
        ci.start()
        return (cu, ci)

    ins = [None] * NCH
    ins[0] = start_in(0)
    prev_out = None
    for k in range(NCH):
        if k + 1 < NCH:
            if prev_out is not None:
                for c in prev_out:
                    c.wait()
                prev_out = None
            ins[k + 1] = start_in(k + 1)
        for c in ins[k]:
            c.wait()
        if prev_out is not None:
            for c in prev_out:
                c.wait()
        prev_out = start_out(k)
    for c in prev_out:
        c.wait()


def kernel(inputs):
    xui = _tc_xui(inputs)
    gu_out, gi_out = _sc_copy(inputs)
    return (xui, gu_out, gi_out)


# speedup vs baseline: 17.5306x; 1.1412x over previous
import functools

import jax
import jax.numpy as jnp
from jax import lax
from jax.experimental import pallas as pl
from jax.experimental.pallas import tpu as pltpu
from jax.experimental.pallas import tpu_sc as plsc

B = 16384
D = 128
R = 8192
NB = B // R

NC, NS = 2, 16
NW = NC * NS
RPW = B // NW
CH = 128
NCH = RPW // CH


def _tc_body(x_ref, gu_hbm, xui_ref, sem_u):
    i = pl.program_id(0)
    cu = pltpu.make_async_copy(
        x_ref.at[0], gu_hbm.at[pl.ds(i * R, R), :], sem_u)
    cu.start()
    xui_ref[...] = jnp.sum(x_ref[0] * x_ref[1], axis=1).reshape(R // 128, 128)
    cu.wait()


def _tc_part(inputs):
    gu, xui2d = pl.pallas_call(
        _tc_body,
        grid=(NB,),
        in_specs=[pl.BlockSpec((2, R, D), lambda i: (0, i, 0))],
        out_specs=[
            pl.BlockSpec(memory_space=pl.ANY),
            pl.BlockSpec((R // 128, 128), lambda i: (i, 0)),
        ],
        out_shape=[
            jax.ShapeDtypeStruct((B, D), jnp.float32),
            jax.ShapeDtypeStruct((B // 128, 128), jnp.float32),
        ],
        scratch_shapes=[pltpu.SemaphoreType.DMA],
    )(inputs)
    return xui2d.reshape(B), gu


_sc_mesh = plsc.VectorSubcoreMesh(core_axis_name="c", subcore_axis_name="s")


@functools.partial(
    pl.kernel,
    out_type=jax.ShapeDtypeStruct((B, D), jnp.float32),
    mesh=_sc_mesh,
    scratch_types=[
        pltpu.VMEM((2, CH, D), jnp.float32),
        pltpu.SemaphoreType.DMA,
        pltpu.SemaphoreType.DMA,
        pltpu.SemaphoreType.DMA,
    ],
)
def _sc_copy(x_hbm, gi_hbm, bi, sem_in0, sem_in1, sem_out):
    wid = lax.axis_index("s") * NC + lax.axis_index("c")
    base = wid * RPW
    in_sems = (sem_in0, sem_in1)

    def start_in(k):
        s = k % 2
        ci = pltpu.make_async_copy(
            x_hbm.at[1, pl.ds(base + k * CH, CH), :], bi.at[s], in_sems[s])
        ci.start()
        return ci

    def start_out(k):
        s = k % 2
        ci = pltpu.make_async_copy(
            bi.at[s], gi_hbm.at[pl.ds(base + k * CH, CH), :], sem_out)
        ci.start()
        return ci

    ins = [None] * NCH
    ins[0] = start_in(0)
    prev_out = None
    for k in range(NCH):
        if k + 1 < NCH:
            if prev_out is not None:
                prev_out.wait()
                prev_out = None
            ins[k + 1] = start_in(k + 1)
        ins[k].wait()
        if prev_out is not None:
            prev_out.wait()
        prev_out = start_out(k)
    prev_out.wait()


def kernel(inputs):
    xui, gu_out = _tc_part(inputs)
    gi_out = _sc_copy(inputs)
    return (xui, gu_out, gi_out)


# speedup vs baseline: 46.3348x; 2.6431x over previous
import jax
import jax.numpy as jnp
from jax.experimental import pallas as pl

B = 16384
D = 128
R = 8192
NB = B // R


def _body(x_ref, gu_ref, gi_ref, xui_ref):
    gu = x_ref[0]
    gi = x_ref[1]
    gu_ref[...] = gu
    gi_ref[...] = gi
    xui_ref[...] = jnp.sum(gu * gi, axis=1).reshape(R // 128, 128)


def kernel(inputs):
    gu_out, gi_out, xui2d = pl.pallas_call(
        _body,
        grid=(NB,),
        in_specs=[pl.BlockSpec((2, R, D), lambda i: (0, i, 0))],
        out_specs=[
            pl.BlockSpec((R, D), lambda i: (i, 0)),
            pl.BlockSpec((R, D), lambda i: (i, 0)),
            pl.BlockSpec((R // 128, 128), lambda i: (i, 0)),
        ],
        out_shape=[
            jax.ShapeDtypeStruct((B, D), jnp.float32),
            jax.ShapeDtypeStruct((B, D), jnp.float32),
            jax.ShapeDtypeStruct((B // 128, 128), jnp.float32),
        ],
    )(inputs)
    return (xui2d.reshape(B), gu_out, gi_out)
